# baseline (speedup 1.0000x reference)
import functools

import jax
import jax.numpy as jnp
from jax import lax
from jax.experimental import pallas as pl
from jax.experimental.pallas import tpu as pltpu
from jax.experimental.pallas import tpu_sc as plsc

_LANES = 128


def kernel(indices, emb_weight):
    (B,) = indices.shape
    V, D = emb_weight.shape

    info = plsc.get_sparse_core_info()
    NC, NS = info.num_cores, info.num_subcores
    NW = NC * NS
    assert B % NW == 0
    b_per_w = B // NW

    mesh = plsc.VectorSubcoreMesh(core_axis_name="c", subcore_axis_name="s")

    NCHUNK = 1
    C = b_per_w // NCHUNK
    NUM_ROWS_PAD = (V + 7) // 8 * 8

    @functools.partial(
        pl.kernel,
        mesh=mesh,
        out_type=jax.ShapeDtypeStruct((B, _LANES), jnp.float32),
        scratch_types=[
            pltpu.VMEM((b_per_w,), jnp.int32),
            pltpu.VMEM((b_per_w, _LANES), jnp.float32),
            pltpu.VMEM_SHARED((NUM_ROWS_PAD, _LANES), jnp.float32),
            [pltpu.SemaphoreType.DMA] * NCHUNK,
            pltpu.SemaphoreType.DMA,
        ],
    )
    def emb_lookup(idx_hbm, table_hbm, out_hbm, idx_v, rows_v, table_sp, gsems, wsem):
        wid = lax.axis_index("s") * NC + lax.axis_index("c")
        base = wid * b_per_w
        @pl.when(lax.axis_index("s") == 0)
        def _stage():
            pltpu.sync_copy(table_hbm, table_sp)

        pltpu.sync_copy(idx_hbm.at[pl.ds(base, b_per_w)], idx_v)
        plsc.subcore_barrier()
        gathers = [
            pltpu.async_copy(
                table_sp.at[idx_v.at[pl.ds(i * C, C)]],
                rows_v.at[pl.ds(i * C, C)],
                gsems[i],
            )
            for i in range(NCHUNK)
        ]
        writes = []
        for i in range(NCHUNK):
            gathers[i].wait()
            writes.append(
                pltpu.async_copy(
                    rows_v.at[pl.ds(i * C, C)],
                    out_hbm.at[pl.ds(base + i * C, C)],
                    wsem,
                )
            )
        for w in writes:
            w.wait()

    table_padded = jnp.pad(emb_weight, ((0, 0), (0, _LANES - D)))
    out = emb_lookup(indices.astype(jnp.int32), table_padded)
    return out[:, :D]


# speedup vs baseline: 1.0216x; 1.0216x over previous
import functools

import jax
import jax.numpy as jnp
from jax import lax
from jax.experimental import pallas as pl
from jax.experimental.pallas import tpu as pltpu
from jax.experimental.pallas import tpu_sc as plsc

_LANES = 128


def kernel(indices, emb_weight):
    (B,) = indices.shape
    V, D = emb_weight.shape

    info = plsc.get_sparse_core_info()
    NC, NS = info.num_cores, info.num_subcores
    NW = NC * NS
    assert B % NW == 0
    b_per_w = B // NW

    mesh = plsc.VectorSubcoreMesh(core_axis_name="c", subcore_axis_name="s")

    NCHUNK = 8
    C = b_per_w // NCHUNK
    NUM_ROWS_PAD = (V + 7) // 8 * 8

    @functools.partial(
        pl.kernel,
        mesh=mesh,
        out_type=jax.ShapeDtypeStruct((B, _LANES), jnp.float32),
        scratch_types=[
            pltpu.VMEM((b_per_w,), jnp.int32),
            pltpu.VMEM((b_per_w, _LANES), jnp.float32),
            pltpu.VMEM_SHARED((NUM_ROWS_PAD, _LANES), jnp.float32),
            [pltpu.SemaphoreType.DMA] * NCHUNK,
            pltpu.SemaphoreType.DMA,
        ],
    )
    def emb_lookup(idx_hbm, table_hbm, out_hbm, idx_v, rows_v, table_sp, gsems, wsem):
        wid = lax.axis_index("s") * NC + lax.axis_index("c")
        base = wid * b_per_w
        @pl.when(lax.axis_index("s") == 0)
        def _stage():
            pltpu.sync_copy(table_hbm, table_sp)

        pltpu.sync_copy(idx_hbm.at[pl.ds(base, b_per_w)], idx_v)
        plsc.subcore_barrier()
        gathers = [
            pltpu.async_copy(
                table_sp.at[idx_v.at[pl.ds(i * C, C)]],
                rows_v.at[pl.ds(i * C, C)],
                gsems[i],
            )
            for i in range(NCHUNK)
        ]
        writes = []
        for i in range(NCHUNK):
            gathers[i].wait()
            writes.append(
                pltpu.async_copy(
                    rows_v.at[pl.ds(i * C, C)],
                    out_hbm.at[pl.ds(base + i * C, C)],
                    wsem,
                )
            )
        for w in writes:
            w.wait()

    table_padded = jnp.pad(emb_weight, ((0, 0), (0, _LANES - D)))
    out = emb_lookup(indices.astype(jnp.int32), table_padded)
    return out[:, :D]
